# baseline (speedup 1.0000x reference)
import functools

import jax
import jax.numpy as jnp
from jax import lax
from jax.experimental import pallas as pl
from jax.experimental.pallas import tpu as pltpu
from jax.experimental.pallas import tpu_sc as plsc

_B, _S, _D = 4, 4096, 1024
_NW = 32
_RPW = _S // _NW
_PCH = 64
_ICH = 32


def _sc_body(x_hbm, p_hbm, o_hbm, pos_v, in_v, sem):
    wid = lax.axis_index("s") * 2 + lax.axis_index("c")
    for c in range(_RPW // _PCH):
        prow = wid * _RPW + c * _PCH
        pltpu.sync_copy(p_hbm.at[pl.ds(prow * _D, _PCH * _D)], pos_v)
        for b in range(_B):
            for s in range(_PCH // _ICH):
                off = (b * _S + prow + s * _ICH) * _D
                pltpu.async_copy(x_hbm.at[pl.ds(off, _ICH * _D)], in_v, sem).wait()

                def add_grp(j, _):
                    g = j * 16
                    in_v[pl.ds(g, 16)] = (
                        in_v[pl.ds(g, 16)] + pos_v[pl.ds(s * _ICH * _D + g, 16)]
                    )
                    return 0

                lax.fori_loop(0, _ICH * _D // 16, add_grp, 0, unroll=8)
                pltpu.sync_copy(in_v, o_hbm.at[pl.ds(off, _ICH * _D)])


def kernel(inputs, pos_encoding):
    B, S, D = inputs.shape
    pos = pos_encoding[:S]
    mesh = plsc.VectorSubcoreMesh(core_axis_name="c", subcore_axis_name="s")
    run = functools.partial(
        pl.kernel,
        mesh=mesh,
        out_type=jax.ShapeDtypeStruct((B * S * D,), jnp.float32),
        scratch_types=[
            pltpu.VMEM((_PCH * _D,), jnp.float32),
            pltpu.VMEM((_ICH * _D,), jnp.float32),
            pltpu.SemaphoreType.DMA,
        ],
    )(_sc_body)
    out = run(inputs.reshape(-1), pos.reshape(-1))
    return out.reshape(B, S, D)


# speedup vs baseline: 1.3072x; 1.3072x over previous
import functools

import jax
import jax.numpy as jnp
from jax import lax
from jax.experimental import pallas as pl
from jax.experimental.pallas import tpu as pltpu
from jax.experimental.pallas import tpu_sc as plsc

_B, _S, _D = 4, 4096, 1024
_NW = 32
_RPW = _S // _NW
_PCH = 32
_ICH = 16
_NCH = _RPW // _PCH
_SUB = _PCH // _ICH


def _sc_body(x_hbm, p_hbm, o_hbm, pos_v, in_v, psem, isem, osem):
    wid = lax.axis_index("s") * 2 + lax.axis_index("c")
    base = wid * _RPW

    def pos_copy(c, pb):
        prow = base + c * _PCH
        return pltpu.async_copy(
            p_hbm.at[pl.ds(prow * _D, _PCH * _D)], pos_v.at[pb], psem)

    def io_off(c, b, s):
        return (b * _S + base + c * _PCH + s * _ICH) * _D

    def in_copy(c, b, s, ib):
        return pltpu.async_copy(
            x_hbm.at[pl.ds(io_off(c, b, s), _ICH * _D)], in_v.at[ib], isem)

    def out_copy(c, b, s, ib):
        return pltpu.async_copy(
            in_v.at[ib], o_hbm.at[pl.ds(io_off(c, b, s), _ICH * _D)], osem)

    iters = [(c, b, s) for c in range(_NCH) for b in range(_B) for s in range(_SUB)]
    n = len(iters)
    pos_h, in_h, out_h = {}, {}, {}
    pos_h[0] = pos_copy(0, 0)
    if _NCH > 1:
        pos_h[1] = pos_copy(1, 1)
    in_h[0] = in_copy(*iters[0], 0)

    for g, (c, b, s) in enumerate(iters):
        ib = g % 2
        pb = c % 2
        in_h[g].wait()
        if b == 0 and s == 0:
            pos_h[c].wait()
        if g + 1 < n:
            if g >= 1:
                out_h[g - 1].wait()
            in_h[g + 1] = in_copy(*iters[g + 1], (g + 1) % 2)

        @plsc.parallel_loop(0, _ICH * _D, step=16, unroll=8)
        def add_grp(i):
            in_v[ib, pl.ds(i, 16)] = (
                in_v[ib, pl.ds(i, 16)] + pos_v[pb, pl.ds(s * _ICH * _D + i, 16)]
            )

        out_h[g] = out_copy(c, b, s, ib)
        if b == _B - 1 and s == _SUB - 1 and c + 2 < _NCH:
            pos_h[c + 2] = pos_copy(c + 2, pb)

    out_h[n - 2].wait()
    out_h[n - 1].wait()


def kernel(inputs, pos_encoding):
    B, S, D = inputs.shape
    pos = pos_encoding[:S]
    mesh = plsc.VectorSubcoreMesh(core_axis_name="c", subcore_axis_name="s")
    run = functools.partial(
        pl.kernel,
        mesh=mesh,
        out_type=jax.ShapeDtypeStruct((B * S * D,), jnp.float32),
        scratch_types=[
            pltpu.VMEM((2, _PCH * _D), jnp.float32),
            pltpu.VMEM((2, _ICH * _D), jnp.float32),
            pltpu.SemaphoreType.DMA,
            pltpu.SemaphoreType.DMA,
            pltpu.SemaphoreType.DMA,
        ],
    )(_sc_body)
    out = run(inputs.reshape(-1), pos.reshape(-1))
    return out.reshape(B, S, D)


# speedup vs baseline: 3.6093x; 2.7610x over previous
import functools

import jax
import jax.numpy as jnp
from jax import lax
from jax.experimental import pallas as pl
from jax.experimental.pallas import tpu as pltpu
from jax.experimental.pallas import tpu_sc as plsc

_B, _S, _D = 4, 4096, 1024
_NW = 32
_RPW = _S // _NW
_PCH = 32
_ICH = 16
_NCH = _RPW // _PCH
_SUB = _PCH // _ICH


def _sc_body(x_hbm, p_hbm, o_hbm, pos_v, in_v, psem, isem, osem):
    wid = lax.axis_index("s") * 2 + lax.axis_index("c")
    base = wid * _RPW

    def pos_copy(c, pb):
        prow = base + c * _PCH
        return pltpu.async_copy(
            p_hbm.at[pl.ds(prow, _PCH), :], pos_v.at[pb], psem)

    def io_row(c, b, s):
        return b * _S + base + c * _PCH + s * _ICH

    def in_copy(c, b, s, ib):
        return pltpu.async_copy(
            x_hbm.at[pl.ds(io_row(c, b, s), _ICH), :], in_v.at[ib], isem)

    def out_copy(c, b, s, ib):
        return pltpu.async_copy(
            in_v.at[ib], o_hbm.at[pl.ds(io_row(c, b, s), _ICH), :], osem)

    iters = [(c, b, s) for c in range(_NCH) for b in range(_B) for s in range(_SUB)]
    n = len(iters)
    pos_h, in_h, out_h = {}, {}, {}
    pos_h[0] = pos_copy(0, 0)
    if _NCH > 1:
        pos_h[1] = pos_copy(1, 1)
    in_h[0] = in_copy(*iters[0], 0)

    for g, (c, b, s) in enumerate(iters):
        ib = g % 2
        pb = c % 2
        in_h[g].wait()
        if b == 0 and s == 0:
            pos_h[c].wait()
        if g + 1 < n:
            if g >= 1:
                out_h[g - 1].wait()
            in_h[g + 1] = in_copy(*iters[g + 1], (g + 1) % 2)

        @plsc.parallel_loop(0, _ICH * _D, step=16, unroll=8)
        def add_grp(i):
            r = i >> 10
            col = pl.multiple_of(i & (_D - 1), 16)
            in_v[ib, r, pl.ds(col, 16)] = (
                in_v[ib, r, pl.ds(col, 16)]
                + pos_v[pb, s * _ICH + r, pl.ds(col, 16)]
            )

        out_h[g] = out_copy(c, b, s, ib)
        if b == _B - 1 and s == _SUB - 1 and c + 2 < _NCH:
            pos_h[c + 2] = pos_copy(c + 2, pb)

    out_h[n - 2].wait()
    out_h[n - 1].wait()


def kernel(inputs, pos_encoding):
    B, S, D = inputs.shape
    pos = pos_encoding[:S]
    mesh = plsc.VectorSubcoreMesh(core_axis_name="c", subcore_axis_name="s")
    run = functools.partial(
        pl.kernel,
        mesh=mesh,
        out_type=jax.ShapeDtypeStruct((B * S, D), jnp.float32),
        scratch_types=[
            pltpu.VMEM((2, _PCH, _D), jnp.float32),
            pltpu.VMEM((2, _ICH, _D), jnp.float32),
            pltpu.SemaphoreType.DMA,
            pltpu.SemaphoreType.DMA,
            pltpu.SemaphoreType.DMA,
        ],
    )(_sc_body)
    out = run(inputs.reshape(B * S, D), pos)
    return out.reshape(B, S, D)
